# Initial kernel scaffold; baseline (speedup 1.0000x reference)
import jax
import jax.numpy as jnp
from jax.experimental import pallas as pl


def kernel(x, edge_index, W1, b1, W2, b2, Wfc, bfc):
    raise NotImplementedError("write your pallas kernel here")


# speedup vs baseline: 32.4827x; 32.4827x over previous
import functools

import jax
import jax.numpy as jnp
from jax import lax
from jax.experimental import pallas as pl
from jax.experimental.pallas import tpu as pltpu
from jax.experimental.pallas import tpu_sc as plsc

N_NODES = 10000
N_EDGES = 320000
IN_DIM = 128
HID = 16

NC = 2
NS = 16
NW = NC * NS

CHUNK = 128
EPW = N_EDGES // NW
CPW = (EPW + CHUNK - 1) // CHUNK
E_PAD = NW * CPW * CHUNK

ACC_ROWS = 10240
STRIP = ACC_ROWS // NS
DUMMY = N_NODES

_mesh = plsc.VectorSubcoreMesh(core_axis_name="c", subcore_axis_name="s")


@functools.partial(
    pl.kernel,
    out_type=jax.ShapeDtypeStruct((NC, ACC_ROWS), jnp.float32),
    mesh=_mesh,
    scratch_types=[
        pltpu.VMEM((CPW, CHUNK), jnp.int32),
        pltpu.VMEM((CHUNK,), jnp.float32),
        pltpu.VMEM((STRIP,), jnp.float32),
        pltpu.VMEM_SHARED((ACC_ROWS,), jnp.float32),
    ],
    compiler_params=pltpu.CompilerParams(use_tc_tiling_on_sc=False),
)
def _sc_counts(dst_hbm, out_hbm, dst_v, ones_v, stage_v, acc_sh):
    cid = lax.axis_index("c")
    sid = lax.axis_index("s")
    wid = sid * NC + cid

    ones16 = jnp.ones((16,), jnp.float32)
    for i in range(CHUNK // 16):
        ones_v[pl.ds(i * 16, 16)] = ones16
    zero16 = jnp.zeros((16,), jnp.float32)

    def zero_body(i, carry):
        stage_v[pl.ds(i * 16, 16)] = zero16
        return carry

    lax.fori_loop(0, STRIP // 16, zero_body, 0)
    pltpu.sync_copy(stage_v, acc_sh.at[pl.ds(sid * STRIP, STRIP)])
    pltpu.sync_copy(dst_hbm.at[wid], dst_v)
    plsc.subcore_barrier()

    def body(j, carry):
        pltpu.sync_copy(ones_v, acc_sh.at[dst_v.at[j]], add=True)
        return carry

    lax.fori_loop(0, CPW, body, 0)
    plsc.subcore_barrier()
    pltpu.sync_copy(
        acc_sh.at[pl.ds(sid * STRIP, STRIP)],
        out_hbm.at[cid, pl.ds(sid * STRIP, STRIP)],
    )


@functools.partial(
    pl.kernel,
    out_type=jax.ShapeDtypeStruct((NC, ACC_ROWS, HID), jnp.float32),
    mesh=_mesh,
    scratch_types=[
        pltpu.VMEM((CPW, CHUNK), jnp.int32),
        pltpu.VMEM((CPW, CHUNK), jnp.int32),
        pltpu.VMEM((CHUNK, HID), jnp.float32),
        pltpu.VMEM_SHARED((ACC_ROWS, HID), jnp.float32),
        pltpu.SemaphoreType.DMA,
    ],
    compiler_params=pltpu.CompilerParams(use_tc_tiling_on_sc=False),
)
def _sc_scatter_rows(vals_hbm, src_hbm, dst_hbm, out_hbm,
                     src_v, dst_v, buf, acc_sh, sem):
    cid = lax.axis_index("c")
    sid = lax.axis_index("s")
    wid = sid * NC + cid

    zero16 = jnp.zeros((16,), jnp.float32)

    def zero_buf(i, carry):
        buf[i, :] = zero16
        return carry

    lax.fori_loop(0, CHUNK, zero_buf, 0)
    for i in range(STRIP // CHUNK):
        pltpu.sync_copy(
            buf, acc_sh.at[pl.ds(sid * STRIP + i * CHUNK, CHUNK)]
        )
    pltpu.sync_copy(src_hbm.at[wid], src_v)
    pltpu.sync_copy(dst_hbm.at[wid], dst_v)
    plsc.subcore_barrier()

    def body(j, carry):
        pltpu.async_copy(vals_hbm.at[src_v.at[j]], buf, sem).wait()
        pltpu.sync_copy(buf, acc_sh.at[dst_v.at[j]], add=True)
        return carry

    lax.fori_loop(0, CPW, body, 0)
    plsc.subcore_barrier()
    pltpu.sync_copy(
        acc_sh.at[pl.ds(sid * STRIP, STRIP)],
        out_hbm.at[cid, pl.ds(sid * STRIP, STRIP)],
    )


def _tc1_body(deg_ref, x_ref, w1_ref, g1_ref):
    dinv = lax.rsqrt(deg_ref[...])
    h = jnp.dot(x_ref[...], w1_ref[...], preferred_element_type=jnp.float32)
    g1_ref[...] = h * dinv


def _tc2_body(p0_ref, p1_ref, g1_ref, deg_ref, w2_ref, b1_ref, g2_ref):
    dinv = lax.rsqrt(deg_ref[...])
    s1 = p0_ref[...] + p1_ref[...] + g1_ref[...]
    out1 = jnp.maximum(s1 * dinv + b1_ref[...], 0.0)
    h2 = jnp.dot(out1, w2_ref[...], preferred_element_type=jnp.float32)
    g2_ref[...] = h2 * dinv


def _tc3_body(p0_ref, p1_ref, g2_ref, deg_ref, b2_ref, wfc_ref, bfc_ref, o_ref):
    dinv = lax.rsqrt(deg_ref[...])
    s2 = p0_ref[...] + p1_ref[...] + g2_ref[...]
    out2 = jnp.maximum(s2 * dinv + b2_ref[...], 0.0)
    pooled = jnp.sum(out2, axis=0, keepdims=True) * (1.0 / N_NODES)
    z = jnp.dot(pooled, wfc_ref[...], preferred_element_type=jnp.float32)
    o_ref[...] = jax.nn.sigmoid(z + bfc_ref[...])


def kernel(x, edge_index, W1, b1, W2, b2, Wfc, bfc):
    src = edge_index[0].astype(jnp.int32)
    dst = edge_index[1].astype(jnp.int32)
    pad = E_PAD - N_EDGES
    src3 = jnp.concatenate(
        [src, jnp.zeros((pad,), jnp.int32)]).reshape(NW, CPW, CHUNK)
    dst3 = jnp.concatenate(
        [dst, jnp.full((pad,), DUMMY, jnp.int32)]).reshape(NW, CPW, CHUNK)

    counts = _sc_counts(dst3)
    deg = (counts[0, :N_NODES] + counts[1, :N_NODES] + 1.0)[:, None]

    g1 = pl.pallas_call(
        _tc1_body,
        out_shape=jax.ShapeDtypeStruct((N_NODES, HID), jnp.float32),
    )(deg, x, W1)

    p1 = _sc_scatter_rows(g1, src3, dst3)

    g2 = pl.pallas_call(
        _tc2_body,
        out_shape=jax.ShapeDtypeStruct((N_NODES, HID), jnp.float32),
    )(p1[0, :N_NODES], p1[1, :N_NODES], g1, deg, W2, b1.reshape(1, HID))

    p2 = _sc_scatter_rows(g2, src3, dst3)

    out = pl.pallas_call(
        _tc3_body,
        out_shape=jax.ShapeDtypeStruct((1, 1), jnp.float32),
    )(p2[0, :N_NODES], p2[1, :N_NODES], g2, deg, b2.reshape(1, HID),
      Wfc, bfc.reshape(1, 1))
    return out.reshape(1)
